# baseline (speedup 1.0000x reference)
import functools

import jax
import jax.numpy as jnp
from jax import lax
from jax.experimental import pallas as pl
from jax.experimental.pallas import tpu as pltpu
from jax.experimental.pallas import tpu_sc as plsc

N = 10000
E = 320000
D = 128
G = 64
L = 3
OUT = D * L

NPAD = 10240
BLK = 1024
NBLK = NPAD // BLK

DH = D // 2
NTILES = 16
EPT = 20480
EPAD = NTILES * EPT
CH = 128
NCHUNK = EPT // CH
ROWS_PER_TILE = NPAD // 16

KROW = 2
KIDX = 8
UNROLL = 8


def _segment_sum_sc_body(h_hbm, idx_hbm, out_hbm,
                         idx_v, rows_v, zb, agg_sh, *sems):
    c = lax.axis_index("c")
    s = lax.axis_index("s")
    wid = s
    gsem = sems[:KROW]
    ssem = sems[KROW:2 * KROW]
    isem = sems[2 * KROW:]

    for q in range(KIDX - 1):
        pltpu.async_copy(idx_hbm.at[wid, q], idx_v.at[q], isem[q])
    pltpu.make_async_copy(idx_hbm.at[wid, 0], idx_v.at[0], isem[0]).wait()
    pltpu.async_copy(h_hbm.at[idx_v.at[0, 0]], rows_v.at[0], gsem[0])

    zero16 = jnp.zeros((16,), jnp.float32)
    for r in range(16):
        for k in range(D // 16):
            zb[r, pl.ds(k * 16, 16)] = zero16

    row0 = s * ROWS_PER_TILE

    def _zero(k, carry):
        pltpu.sync_copy(zb, agg_sh.at[pl.ds(row0 + k * 16, 16)])
        return carry

    lax.fori_loop(0, ROWS_PER_TILE // 16, _zero, 0)
    plsc.subcore_barrier()

    def _edges(j0, carry):
        for v in range(UNROLL):
            b = v % KROW
            u = v % KIDX
            b1 = (v + 1) % KROW
            u1 = (v + 1) % KIDX
            uf = (v - 1) % KIDX
            j = j0 * UNROLL + v
            pltpu.make_async_copy(h_hbm.at[idx_v.at[u, 0]], rows_v.at[b],
                                  gsem[b]).wait()
            pltpu.async_copy(rows_v.at[b], agg_sh.at[idx_v.at[u, 1]],
                             ssem[b], add=True)

            @pl.when(j >= 1)
            def _():
                pltpu.make_async_copy(rows_v.at[b1],
                                      agg_sh.at[idx_v.at[u1, 1]],
                                      ssem[b1]).wait()

            @pl.when(j + 7 < NCHUNK)
            def _():
                pltpu.async_copy(idx_hbm.at[wid, j + 7], idx_v.at[uf],
                                 isem[uf])

            @pl.when(j + 1 < NCHUNK)
            def _():
                pltpu.make_async_copy(idx_hbm.at[wid, u1], idx_v.at[u1],
                                      isem[u1]).wait()
                pltpu.async_copy(h_hbm.at[idx_v.at[u1, 0]], rows_v.at[b1],
                                 gsem[b1])
        return carry

    lax.fori_loop(0, NCHUNK // UNROLL, _edges, 0)
    j = NCHUNK - 1
    pltpu.make_async_copy(rows_v.at[j % KROW],
                          agg_sh.at[idx_v.at[j % KIDX, 1]],
                          ssem[j % KROW]).wait()
    plsc.subcore_barrier()

    pltpu.sync_copy(agg_sh.at[pl.ds(row0, ROWS_PER_TILE)],
                    out_hbm.at[c, pl.ds(row0, ROWS_PER_TILE)])


@functools.cache
def _get_segment_sum_sc():
    mesh = plsc.VectorSubcoreMesh(core_axis_name="c", subcore_axis_name="s")
    return pl.kernel(
        _segment_sum_sc_body,
        mesh=mesh,
        out_type=jax.ShapeDtypeStruct((2, NPAD, D), jnp.float32),
        scratch_types=[
            pltpu.VMEM((KIDX, 2, CH), jnp.int32),
            pltpu.VMEM((KROW, CH, D), jnp.float32),
            pltpu.VMEM((16, D), jnp.float32),
            pltpu.VMEM_SHARED((NPAD, D), jnp.float32),
        ] + [pltpu.SemaphoreType.DMA] * (2 * KROW + KIDX),
    )


def _mlp_body(h_ref, agg_ref, w1_ref, b1_ref, w2_ref, b2_ref,
              v_ref, sums_ref):
    i = pl.program_id(0)
    z = h_ref[...] + agg_ref[0]
    u = lax.dot_general(z, w1_ref[...], (((1,), (1,)), ((), ())),
                        preferred_element_type=jnp.float32) + b1_ref[...]
    u = jnp.maximum(u, 0.0)
    v = lax.dot_general(u, w2_ref[...], (((1,), (1,)), ((), ())),
                        preferred_element_type=jnp.float32) + b2_ref[...]
    v = jnp.maximum(v, 0.0)
    rows = i * BLK + lax.broadcasted_iota(jnp.int32, (BLK, 1), 0)
    v = jnp.where(rows < N, v, 0.0)
    v_ref[...] = v
    s0 = jnp.sum(v, axis=0, keepdims=True)
    s1 = jnp.sum(v * v, axis=0, keepdims=True)
    upd = jnp.concatenate([s0, s1, jnp.zeros((6, D), jnp.float32)], axis=0)

    @pl.when(i == 0)
    def _():
        sums_ref[...] = jnp.zeros_like(sums_ref)

    sums_ref[...] += upd


_mlp = pl.pallas_call(
    _mlp_body,
    grid=(NBLK,),
    in_specs=[
        pl.BlockSpec((BLK, D), lambda i: (i, 0)),
        pl.BlockSpec((1, BLK, D), lambda i: (0, i, 0)),
        pl.BlockSpec((D, D), lambda i: (0, 0)),
        pl.BlockSpec((1, D), lambda i: (0, 0)),
        pl.BlockSpec((D, D), lambda i: (0, 0)),
        pl.BlockSpec((1, D), lambda i: (0, 0)),
    ],
    out_specs=[
        pl.BlockSpec((BLK, D), lambda i: (i, 0)),
        pl.BlockSpec((8, D), lambda i: (0, 0)),
    ],
    out_shape=[
        jax.ShapeDtypeStruct((NPAD, D), jnp.float32),
        jax.ShapeDtypeStruct((8, D), jnp.float32),
    ],
)


def _bn_pool_body(v_ref, sums_ref, g_ref, bt_ref, batch_ref, z_ref, pool_ref):
    i = pl.program_id(0)
    inv_n = 1.0 / N
    mean = sums_ref[0:1, :] * inv_n
    var = sums_ref[1:2, :] * inv_n - mean * mean
    rstd = lax.rsqrt(var + 1e-5)
    z = (v_ref[...] - mean) * rstd * g_ref[...] + bt_ref[...]
    z_ref[...] = z
    b_row = batch_ref[0]
    gids = lax.broadcasted_iota(jnp.int32, (G, 1), 0)
    oh = (gids == b_row).astype(jnp.float32)
    pp = lax.dot_general(oh, z, (((1,), (0,)), ((), ())),
                         precision=lax.Precision.HIGHEST,
                         preferred_element_type=jnp.float32)

    @pl.when(i == 0)
    def _():
        pool_ref[...] = jnp.zeros_like(pool_ref)

    pool_ref[...] += pp


_bn_pool = pl.pallas_call(
    _bn_pool_body,
    grid=(NBLK,),
    in_specs=[
        pl.BlockSpec((BLK, D), lambda i: (i, 0)),
        pl.BlockSpec((8, D), lambda i: (0, 0)),
        pl.BlockSpec((1, D), lambda i: (0, 0)),
        pl.BlockSpec((1, D), lambda i: (0, 0)),
        pl.BlockSpec((1, 1, BLK), lambda i: (i, 0, 0)),
    ],
    out_specs=[
        pl.BlockSpec((BLK, D), lambda i: (i, 0)),
        pl.BlockSpec((G, D), lambda i: (0, 0)),
    ],
    out_shape=[
        jax.ShapeDtypeStruct((NPAD, D), jnp.float32),
        jax.ShapeDtypeStruct((G, D), jnp.float32),
    ],
)


def _head_body(p_ref, w1_ref, b1_ref, w2_ref, b2_ref, yn_ref, xn_ref):
    p = p_ref[...]
    t = lax.dot_general(p, w1_ref[...], (((1,), (1,)), ((), ())),
                        preferred_element_type=jnp.float32) + b1_ref[...]
    t = jnp.maximum(t, 0.0)
    y = lax.dot_general(t, w2_ref[...], (((1,), (1,)), ((), ())),
                        preferred_element_type=jnp.float32) + b2_ref[...]
    pn = jnp.sqrt(jnp.sum(p * p, axis=1, keepdims=True))
    yn = jnp.sqrt(jnp.sum(y * y, axis=1, keepdims=True))
    xn_ref[...] = p / jnp.maximum(pn, 1e-12)
    yn_ref[...] = y / jnp.maximum(yn, 1e-12)


_head = pl.pallas_call(
    _head_body,
    out_shape=[
        jax.ShapeDtypeStruct((G, OUT), jnp.float32),
        jax.ShapeDtypeStruct((G, OUT), jnp.float32),
    ],
)


def kernel(x, edge_index, batch, W1, b1, W2, b2, gamma, beta, Wp1, bp1, Wp2, bp2):
    f32 = jnp.float32
    src = edge_index[0].astype(jnp.int32)
    dst = edge_index[1].astype(jnp.int32)
    n_extra = NPAD - N
    ppt = EPT - E // NTILES
    pad_dst = N + (jnp.arange(ppt, dtype=jnp.int32) % n_extra)
    src_p = jnp.concatenate(
        [src.reshape(NTILES, E // NTILES),
         jnp.zeros((NTILES, ppt), jnp.int32)], axis=1
    ).reshape(NTILES, NCHUNK, CH)
    dst_p = jnp.concatenate(
        [dst.reshape(NTILES, E // NTILES),
         jnp.broadcast_to(pad_dst, (NTILES, ppt))], axis=1
    ).reshape(NTILES, NCHUNK, CH)
    idx_il = jnp.stack([src_p, dst_p], axis=2)
    h = jnp.concatenate([x.astype(f32), jnp.zeros((n_extra, D), f32)], axis=0)
    batch_p = jnp.concatenate(
        [batch.astype(jnp.int32), jnp.full((n_extra,), G, jnp.int32)]
    ).reshape(NBLK, 1, BLK)

    seg_sum = _get_segment_sum_sc()
    pools = []
    for i in range(L):
        agg = seg_sum(h, idx_il)
        v, sums = _mlp(h, agg, W1[i], b1[i][None, :], W2[i], b2[i][None, :])
        h, pool = _bn_pool(v, sums, gamma[i][None, :], beta[i][None, :],
                           batch_p)
        pools.append(pool)
    pooled = jnp.concatenate(pools, axis=1)
    yn, xn = _head(pooled, Wp1, bp1[None, :], Wp2, bp2[None, :])
    return (yn, xn)


# speedup vs baseline: 1.9835x; 1.9835x over previous
import functools

import jax
import jax.numpy as jnp
from jax import lax
from jax.experimental import pallas as pl
from jax.experimental.pallas import tpu as pltpu
from jax.experimental.pallas import tpu_sc as plsc

N = 10000
E = 320000
D = 128
G = 64
L = 3
OUT = D * L

NPAD = 10240
BLK = 1024
NBLK = NPAD // BLK

NTILES = 32
EPT = 10240
EPAD = NTILES * EPT
CH = 128
NCHUNK = EPT // CH
ROWS_PER_TILE = NPAD // 16

KROW = 2
KIDX = 4


def _segment_sum_sc_body(h_hbm, idx_hbm, out_hbm,
                         idx_v, rows_v, zb, agg_sh, *sems):
    c = lax.axis_index("c")
    s = lax.axis_index("s")
    wid = c * 16 + s
    gsem = sems[:KROW]
    isem = sems[KROW:]

    for q in range(KIDX):
        pltpu.async_copy(idx_hbm.at[wid, q], idx_v.at[q], isem[q])
    for b in range(KROW):
        pltpu.make_async_copy(idx_hbm.at[wid, b], idx_v.at[b],
                              isem[b]).wait()
        pltpu.async_copy(h_hbm.at[idx_v.at[b, 0]], rows_v.at[b], gsem[b])

    zero16 = jnp.zeros((16,), jnp.float32)
    for r in range(16):
        for k in range(D // 16):
            zb[r, pl.ds(k * 16, 16)] = zero16

    row0 = s * ROWS_PER_TILE

    def _zero(k, carry):
        pltpu.sync_copy(zb, agg_sh.at[pl.ds(row0 + k * 16, 16)])
        return carry

    lax.fori_loop(0, ROWS_PER_TILE // 16, _zero, 0)
    plsc.subcore_barrier()

    def _edges(j0, carry):
        for u in range(KIDX):
            j = j0 * KIDX + u
            b = u % KROW
            q2 = (u + KROW) % KIDX
            pltpu.make_async_copy(h_hbm.at[idx_v.at[u, 0]], rows_v.at[b],
                                  gsem[b]).wait()
            pltpu.sync_copy(rows_v.at[b], agg_sh.at[idx_v.at[u, 1]],
                            add=True)

            @pl.when(j + KIDX < NCHUNK)
            def _():
                pltpu.async_copy(idx_hbm.at[wid, j + KIDX], idx_v.at[u],
                                 isem[u])

            @pl.when(j + KROW < NCHUNK)
            def _():
                pltpu.make_async_copy(idx_hbm.at[wid, u], idx_v.at[q2],
                                      isem[q2]).wait()
                pltpu.async_copy(h_hbm.at[idx_v.at[q2, 0]], rows_v.at[b],
                                 gsem[b])
        return carry

    lax.fori_loop(0, NCHUNK // KIDX, _edges, 0)
    plsc.subcore_barrier()

    pltpu.sync_copy(agg_sh.at[pl.ds(row0, ROWS_PER_TILE)],
                    out_hbm.at[c, pl.ds(row0, ROWS_PER_TILE)])


@functools.cache
def _get_segment_sum_sc():
    mesh = plsc.VectorSubcoreMesh(core_axis_name="c", subcore_axis_name="s")
    return pl.kernel(
        _segment_sum_sc_body,
        mesh=mesh,
        out_type=jax.ShapeDtypeStruct((2, NPAD, D), jnp.float32),
        scratch_types=[
            pltpu.VMEM((KIDX, 2, CH), jnp.int32),
            pltpu.VMEM((KROW, CH, D), jnp.float32),
            pltpu.VMEM((16, D), jnp.float32),
            pltpu.VMEM_SHARED((NPAD, D), jnp.float32),
        ] + [pltpu.SemaphoreType.DMA] * (KROW + KIDX),
    )


def _mlp_body(h_ref, agg_ref, w1_ref, b1_ref, w2_ref, b2_ref,
              v_ref, sums_ref):
    i = pl.program_id(0)
    z = h_ref[...] + agg_ref[0] + agg_ref[1]
    u = lax.dot_general(z, w1_ref[...], (((1,), (1,)), ((), ())),
                        preferred_element_type=jnp.float32) + b1_ref[...]
    u = jnp.maximum(u, 0.0)
    v = lax.dot_general(u, w2_ref[...], (((1,), (1,)), ((), ())),
                        preferred_element_type=jnp.float32) + b2_ref[...]
    v = jnp.maximum(v, 0.0)
    rows = i * BLK + lax.broadcasted_iota(jnp.int32, (BLK, 1), 0)
    v = jnp.where(rows < N, v, 0.0)
    v_ref[...] = v
    s0 = jnp.sum(v, axis=0, keepdims=True)
    s1 = jnp.sum(v * v, axis=0, keepdims=True)
    upd = jnp.concatenate([s0, s1, jnp.zeros((6, D), jnp.float32)], axis=0)

    @pl.when(i == 0)
    def _():
        sums_ref[...] = jnp.zeros_like(sums_ref)

    sums_ref[...] += upd


_mlp = pl.pallas_call(
    _mlp_body,
    grid=(NBLK,),
    in_specs=[
        pl.BlockSpec((BLK, D), lambda i: (i, 0)),
        pl.BlockSpec((2, BLK, D), lambda i: (0, i, 0)),
        pl.BlockSpec((D, D), lambda i: (0, 0)),
        pl.BlockSpec((1, D), lambda i: (0, 0)),
        pl.BlockSpec((D, D), lambda i: (0, 0)),
        pl.BlockSpec((1, D), lambda i: (0, 0)),
    ],
    out_specs=[
        pl.BlockSpec((BLK, D), lambda i: (i, 0)),
        pl.BlockSpec((8, D), lambda i: (0, 0)),
    ],
    out_shape=[
        jax.ShapeDtypeStruct((NPAD, D), jnp.float32),
        jax.ShapeDtypeStruct((8, D), jnp.float32),
    ],
)


def _bn_pool_body(v_ref, sums_ref, g_ref, bt_ref, batch_ref, z_ref, pool_ref):
    i = pl.program_id(0)
    inv_n = 1.0 / N
    mean = sums_ref[0:1, :] * inv_n
    var = sums_ref[1:2, :] * inv_n - mean * mean
    rstd = lax.rsqrt(var + 1e-5)
    z = (v_ref[...] - mean) * rstd * g_ref[...] + bt_ref[...]
    z_ref[...] = z
    b_row = batch_ref[0]
    gids = lax.broadcasted_iota(jnp.int32, (G, 1), 0)
    oh = (gids == b_row).astype(jnp.float32)
    pp = lax.dot_general(oh, z, (((1,), (0,)), ((), ())),
                         precision=lax.Precision.HIGHEST,
                         preferred_element_type=jnp.float32)

    @pl.when(i == 0)
    def _():
        pool_ref[...] = jnp.zeros_like(pool_ref)

    pool_ref[...] += pp


_bn_pool = pl.pallas_call(
    _bn_pool_body,
    grid=(NBLK,),
    in_specs=[
        pl.BlockSpec((BLK, D), lambda i: (i, 0)),
        pl.BlockSpec((8, D), lambda i: (0, 0)),
        pl.BlockSpec((1, D), lambda i: (0, 0)),
        pl.BlockSpec((1, D), lambda i: (0, 0)),
        pl.BlockSpec((1, 1, BLK), lambda i: (i, 0, 0)),
    ],
    out_specs=[
        pl.BlockSpec((BLK, D), lambda i: (i, 0)),
        pl.BlockSpec((G, D), lambda i: (0, 0)),
    ],
    out_shape=[
        jax.ShapeDtypeStruct((NPAD, D), jnp.float32),
        jax.ShapeDtypeStruct((G, D), jnp.float32),
    ],
)


def _head_body(p_ref, w1_ref, b1_ref, w2_ref, b2_ref, yn_ref, xn_ref):
    p = p_ref[...]
    t = lax.dot_general(p, w1_ref[...], (((1,), (1,)), ((), ())),
                        preferred_element_type=jnp.float32) + b1_ref[...]
    t = jnp.maximum(t, 0.0)
    y = lax.dot_general(t, w2_ref[...], (((1,), (1,)), ((), ())),
                        preferred_element_type=jnp.float32) + b2_ref[...]
    pn = jnp.sqrt(jnp.sum(p * p, axis=1, keepdims=True))
    yn = jnp.sqrt(jnp.sum(y * y, axis=1, keepdims=True))
    xn_ref[...] = p / jnp.maximum(pn, 1e-12)
    yn_ref[...] = y / jnp.maximum(yn, 1e-12)


_head = pl.pallas_call(
    _head_body,
    out_shape=[
        jax.ShapeDtypeStruct((G, OUT), jnp.float32),
        jax.ShapeDtypeStruct((G, OUT), jnp.float32),
    ],
)


def kernel(x, edge_index, batch, W1, b1, W2, b2, gamma, beta, Wp1, bp1, Wp2, bp2):
    f32 = jnp.float32
    src = edge_index[0].astype(jnp.int32)
    dst = edge_index[1].astype(jnp.int32)
    n_extra = NPAD - N
    ppt = EPT - E // NTILES
    pad_dst = N + (jnp.arange(ppt, dtype=jnp.int32) % n_extra)
    src_p = jnp.concatenate(
        [src.reshape(NTILES, E // NTILES),
         jnp.zeros((NTILES, ppt), jnp.int32)], axis=1
    ).reshape(NTILES, NCHUNK, CH)
    dst_p = jnp.concatenate(
        [dst.reshape(NTILES, E // NTILES),
         jnp.broadcast_to(pad_dst, (NTILES, ppt))], axis=1
    ).reshape(NTILES, NCHUNK, CH)
    idx_il = jnp.stack([src_p, dst_p], axis=2)
    h = jnp.concatenate([x.astype(f32), jnp.zeros((n_extra, D), f32)], axis=0)
    batch_p = jnp.concatenate(
        [batch.astype(jnp.int32), jnp.full((n_extra,), G, jnp.int32)]
    ).reshape(NBLK, 1, BLK)

    seg_sum = _get_segment_sum_sc()
    pools = []
    for i in range(L):
        agg = seg_sum(h, idx_il)
        v, sums = _mlp(h, agg, W1[i], b1[i][None, :], W2[i], b2[i][None, :])
        h, pool = _bn_pool(v, sums, gamma[i][None, :], beta[i][None, :],
                           batch_p)
        pools.append(pool)
    pooled = jnp.concatenate(pools, axis=1)
    yn, xn = _head(pooled, Wp1, bp1[None, :], Wp2, bp2[None, :])
    return (yn, xn)
